# baseline (speedup 1.0000x reference)
import functools

import jax
import jax.numpy as jnp
from jax import lax
from jax.experimental import pallas as pl
from jax.experimental.pallas import tpu as pltpu
from jax.experimental.pallas import tpu_sc as plsc

B, L, H, V = 4096, 50, 64, 1000000
N = B * L
NC, NS = 2, 16
NW = NC * NS
CHUNK = 128
ROWS_PW = N // NW
CH_PW = ROWS_PW // CHUNK
EPS = 1e-5
GRPS = CHUNK // 16


def _rsqrt(x):
    i = plsc.bitcast(x, jnp.int32)
    i = 0x5F3759DF - (i >> 1)
    y = plsc.bitcast(i, jnp.float32)
    for _ in range(3):
        y = y * (1.5 - (0.5 * x) * y * y)
    return y


def _ln_chunk(buf, gamma_v, beta_v):
    iota = lax.iota(jnp.int32, 16)
    grp_rows = [iota + 16 * g for g in range(GRPS)]
    zero = jnp.zeros((16,), jnp.float32)

    def pass_a(h, accs):
        col = jnp.full((16,), h, jnp.int32)
        out = []
        for g in range(GRPS):
            x = plsc.load_gather(buf, [grp_rows[g], col])
            out.append(accs[2 * g] + x)
            out.append(accs[2 * g + 1] + x * x)
        return tuple(out)

    accs = lax.fori_loop(0, H, pass_a, (zero,) * (2 * GRPS))
    means, rstds = [], []
    for g in range(GRPS):
        mean = accs[2 * g] * (1.0 / H)
        var = accs[2 * g + 1] * (1.0 / H) - mean * mean + EPS
        means.append(mean)
        rstds.append(_rsqrt(var))

    def pass_b(h, carry):
        col = jnp.full((16,), h, jnp.int32)
        gv = gamma_v[h, :]
        bv = beta_v[h, :]
        for g in range(GRPS):
            x = plsc.load_gather(buf, [grp_rows[g], col])
            y = (x - means[g]) * rstds[g] * gv + bv
            plsc.store_scatter(buf, [grp_rows[g], col], y)
        return carry

    lax.fori_loop(0, H, pass_b, 0)


@functools.partial(
    pl.kernel,
    out_type=(
        jax.ShapeDtypeStruct((N, H), jnp.float32),
        jax.ShapeDtypeStruct((N, H), jnp.float32),
        jax.ShapeDtypeStruct((B, H), jnp.float32),
    ),
    mesh=plsc.VectorSubcoreMesh(core_axis_name="c", subcore_axis_name="s"),
    compiler_params=pltpu.CompilerParams(
        use_tc_tiling_on_sc=False, needs_layout_passes=False),
    scratch_types=[
        pltpu.VMEM((2 * CH_PW, CHUNK), jnp.int32),
        pltpu.VMEM((1, CHUNK), jnp.int32),
        pltpu.VMEM((CHUNK, H), jnp.float32),
        pltpu.VMEM((CHUNK, H), jnp.float32),
        pltpu.VMEM((H, 16), jnp.float32),
        pltpu.VMEM((H, 16), jnp.float32),
        pltpu.SemaphoreType.DMA,
        pltpu.SemaphoreType.DMA,
    ],
)
def _embed_ln(dx_idx, proc_idx, dx_tab, proc_tab, visit_tab, gamma_b, beta_b,
              dx_out, proc_out, visit_out,
              idx_v, zidx_v, buf0, buf1, gamma_v, beta_v, sem0, sem1):
    wid = lax.axis_index("s") * NC + lax.axis_index("c")
    base_row = wid * CH_PW
    pltpu.sync_copy(dx_idx.at[pl.ds(base_row, CH_PW)], idx_v.at[pl.ds(0, CH_PW)])
    pltpu.sync_copy(proc_idx.at[pl.ds(base_row, CH_PW)],
                    idx_v.at[pl.ds(CH_PW, CH_PW)])
    pltpu.sync_copy(gamma_b, gamma_v)
    pltpu.sync_copy(beta_b, beta_v)
    zero_i = jnp.zeros((16,), jnp.int32)
    for i in range(GRPS):
        zidx_v[0, pl.ds(16 * i, 16)] = zero_i

    bufs = (buf0, buf1)
    sems = (sem0, sem1)

    def gather(j, buf, sem):
        tab = jnp.where(j < CH_PW, 0, 1)

        @pl.when(tab == 0)
        def _():
            pltpu.async_copy(dx_tab.at[idx_v.at[j]], buf, sem)

        @pl.when(tab == 1)
        def _():
            pltpu.async_copy(proc_tab.at[idx_v.at[j]], buf, sem)

    def drain(buf, sem):
        pltpu.make_async_copy(dx_tab.at[idx_v.at[0]], buf, sem).wait()

    def flush(j, buf):
        out_base = wid * ROWS_PW

        @pl.when(j < CH_PW)
        def _():
            pltpu.sync_copy(buf, dx_out.at[pl.ds(out_base + j * CHUNK, CHUNK)])

        @pl.when(j >= CH_PW)
        def _():
            pltpu.sync_copy(
                buf, proc_out.at[pl.ds(out_base + (j - CH_PW) * CHUNK, CHUNK)])

    gather(0, bufs[0], sems[0])

    def body(j, carry):
        cur = lax.rem(j, 2)
        for p in range(2):
            @pl.when(cur == p)
            def _():
                @pl.when(j + 1 < 2 * CH_PW)
                def _():
                    gather(j + 1, bufs[1 - p], sems[1 - p])
                drain(bufs[p], sems[p])
                _ln_chunk(bufs[p], gamma_v, beta_v)
                flush(j, bufs[p])
        return carry

    lax.fori_loop(0, 2 * CH_PW, body, 0)

    pltpu.async_copy(visit_tab.at[zidx_v.at[0]], buf0, sem0).wait()
    _ln_chunk(buf0, gamma_v, beta_v)
    pltpu.sync_copy(buf0, visit_out.at[pl.ds(wid * CHUNK, CHUNK)])


def kernel(dx_ints1, proc_ints1, number, dx_table, proc_table, visit_table,
           ln_gamma, ln_beta):
    del number
    batch = dx_ints1.shape[0]
    dx_idx = dx_ints1.reshape(N // CHUNK, CHUNK)
    proc_idx = proc_ints1.reshape(N // CHUNK, CHUNK)
    gamma_b = jnp.broadcast_to(ln_gamma[:, None], (H, 16))
    beta_b = jnp.broadcast_to(ln_beta[:, None], (H, 16))
    dx_o, proc_o, visit_o = _embed_ln(
        dx_idx, proc_idx, dx_table, proc_table, visit_table, gamma_b, beta_b)
    return (
        dx_o.reshape(batch, L, H),
        proc_o.reshape(batch, L, H),
        visit_o.reshape(batch, 1, H),
        jnp.ones((batch, 1), jnp.float32),
    )


# speedup vs baseline: 1.5769x; 1.5769x over previous
import functools

import jax
import jax.numpy as jnp
from jax import lax
from jax.experimental import pallas as pl
from jax.experimental.pallas import tpu as pltpu
from jax.experimental.pallas import tpu_sc as plsc

B, L, H, V = 4096, 50, 64, 1000000
NC, NS = 2, 16
NW = NC * NS
B_PW = B // NW
BCH = 8
CHUNK = BCH * L
NCH = B_PW // BCH
EPS = 1e-5
BLOCKS = ((0, 8), (8, 8), (16, 8), (24, 1))


def _rsqrt(x):
    i = plsc.bitcast(x, jnp.int32)
    i = 0x5F3759DF - (i >> 1)
    y = plsc.bitcast(i, jnp.float32)
    for _ in range(3):
        y = y * (1.5 - (0.5 * x) * y * y)
    return y


def _ln_block(buf, gamma_v, beta_v, g0, ngrp):
    iota = lax.iota(jnp.int32, 16)
    grp_rows = [iota + 16 * (g0 + g) for g in range(ngrp)]
    zero = jnp.zeros((16,), jnp.float32)

    def pass_a(h, accs):
        col = (h + iota) & (H - 1)
        out = []
        for g in range(ngrp):
            x = plsc.load_gather(buf, [grp_rows[g], col])
            out.append(accs[2 * g] + x)
            out.append(accs[2 * g + 1] + x * x)
        return tuple(out)

    accs = lax.fori_loop(0, H, pass_a, (zero,) * (2 * ngrp))
    means, rstds = [], []
    for g in range(ngrp):
        mean = accs[2 * g] * (1.0 / H)
        var = accs[2 * g + 1] * (1.0 / H) - mean * mean + EPS
        means.append(mean)
        rstds.append(_rsqrt(var))

    def pass_b(h, carry):
        col = (h + iota) & (H - 1)
        gv = plsc.load_gather(gamma_v, [col])
        bv = plsc.load_gather(beta_v, [col])
        for g in range(ngrp):
            x = plsc.load_gather(buf, [grp_rows[g], col])
            y = (x - means[g]) * rstds[g] * gv + bv
            plsc.store_scatter(buf, [grp_rows[g], col], y)
        return carry

    lax.fori_loop(0, H, pass_b, 0)


def _ln_chunk(buf, gamma_v, beta_v, blocks=BLOCKS):
    for g0, ngrp in blocks:
        _ln_block(buf, gamma_v, beta_v, g0, ngrp)


@functools.partial(
    pl.kernel,
    out_type=(
        jax.ShapeDtypeStruct((B, L, H), jnp.float32),
        jax.ShapeDtypeStruct((B, L, H), jnp.float32),
        jax.ShapeDtypeStruct((B, H), jnp.float32),
    ),
    mesh=plsc.VectorSubcoreMesh(core_axis_name="c", subcore_axis_name="s"),
    compiler_params=pltpu.CompilerParams(
        use_tc_tiling_on_sc=False, needs_layout_passes=False),
    scratch_types=[
        pltpu.VMEM((2 * B_PW, L), jnp.int32),
        pltpu.VMEM((1, 128), jnp.int32),
        pltpu.VMEM((CHUNK, H), jnp.float32),
        pltpu.VMEM((CHUNK, H), jnp.float32),
        pltpu.VMEM((H,), jnp.float32),
        pltpu.VMEM((H,), jnp.float32),
        pltpu.SemaphoreType.DMA,
        pltpu.SemaphoreType.DMA,
        pltpu.SemaphoreType.DMA,
        pltpu.SemaphoreType.DMA,
    ],
)
def _embed_ln(dx_idx, proc_idx, dx_tab, proc_tab, visit_tab, gamma_b, beta_b,
              dx_out, proc_out, visit_out,
              idx_v, zidx_v, buf0, buf1, gamma_v, beta_v,
              gsem0, gsem1, fsem0, fsem1):
    wid = lax.axis_index("s") * NC + lax.axis_index("c")
    b_base = wid * B_PW
    pltpu.sync_copy(dx_idx.at[pl.ds(b_base, B_PW)], idx_v.at[pl.ds(0, B_PW)])
    pltpu.sync_copy(proc_idx.at[pl.ds(b_base, B_PW)],
                    idx_v.at[pl.ds(B_PW, B_PW)])
    pltpu.sync_copy(gamma_b, gamma_v)
    pltpu.sync_copy(beta_b, beta_v)
    zero_i = jnp.zeros((16,), jnp.int32)
    for i in range(8):
        zidx_v[0, pl.ds(16 * i, 16)] = zero_i

    bufs = (buf0, buf1)
    gsems = (gsem0, gsem1)
    fsems = (fsem0, fsem1)
    total = 2 * NCH

    def gather(j, buf, sem):
        @pl.when(j < NCH)
        def _():
            for i in range(BCH):
                pltpu.async_copy(dx_tab.at[idx_v.at[j * BCH + i]],
                                 buf.at[pl.ds(i * L, L)], sem)

        @pl.when(j >= NCH)
        def _():
            for i in range(BCH):
                pltpu.async_copy(proc_tab.at[idx_v.at[j * BCH + i]],
                                 buf.at[pl.ds(i * L, L)], sem)

    def gather_drain(buf, sem):
        for i in range(BCH):
            pltpu.make_async_copy(dx_tab.at[idx_v.at[0]],
                                  buf.at[pl.ds(i * L, L)], sem).wait()

    def flush(j, buf, sem):
        @pl.when(j < NCH)
        def _():
            for i in range(BCH):
                pltpu.async_copy(buf.at[pl.ds(i * L, L)],
                                 dx_out.at[b_base + j * BCH + i], sem)

        @pl.when(j >= NCH)
        def _():
            for i in range(BCH):
                pltpu.async_copy(buf.at[pl.ds(i * L, L)],
                                 proc_out.at[b_base + (j - NCH) * BCH + i],
                                 sem)

    def flush_drain(buf, sem):
        for i in range(BCH):
            pltpu.make_async_copy(buf.at[pl.ds(i * L, L)], dx_out.at[0],
                                  sem).wait()

    gather(0, bufs[0], gsems[0])

    def body(j, carry):
        cur = lax.rem(j, 2)
        for p in range(2):
            @pl.when(cur == p)
            def _():
                gather_drain(bufs[p], gsems[p])

                @pl.when(j + 1 < total)
                def _():
                    @pl.when(j >= 1)
                    def _():
                        flush_drain(bufs[1 - p], fsems[1 - p])
                    gather(j + 1, bufs[1 - p], gsems[1 - p])

                _ln_chunk(bufs[p], gamma_v, beta_v)
                flush(j, bufs[p], fsems[p])
        return carry

    lax.fori_loop(0, total, body, 0)
    flush_drain(bufs[0], fsems[0])
    flush_drain(bufs[1], fsems[1])

    pltpu.async_copy(visit_tab.at[zidx_v.at[0]],
                     buf0.at[pl.ds(0, 128)], gsem0).wait()
    _ln_chunk(buf0.at[pl.ds(0, 128)], gamma_v, beta_v,
              blocks=((0, 8),))
    pltpu.sync_copy(buf0.at[pl.ds(0, 128)],
                    visit_out.at[pl.ds(wid * 128, 128)])


def kernel(dx_ints1, proc_ints1, number, dx_table, proc_table, visit_table,
           ln_gamma, ln_beta):
    del number
    batch = dx_ints1.shape[0]
    dx_o, proc_o, visit_o = _embed_ln(
        dx_ints1, proc_ints1, dx_table, proc_table, visit_table,
        ln_gamma, ln_beta)
    return (
        dx_o,
        proc_o,
        visit_o[:, None, :],
        jnp.ones((batch, 1), jnp.float32),
    )


# speedup vs baseline: 1.7312x; 1.0979x over previous
import functools

import jax
import jax.numpy as jnp
from jax import lax
from jax.experimental import pallas as pl
from jax.experimental.pallas import tpu as pltpu
from jax.experimental.pallas import tpu_sc as plsc

B, L, H, V = 4096, 50, 64, 1000000
NC, NS = 2, 16
NW = NC * NS
B_PW = B // NW
GRPS = B_PW // 16
NCH = 2 * L + 1
EPS = 1e-5


def _rsqrt(x):
    i = plsc.bitcast(x, jnp.int32)
    i = 0x5F3759DF - (i >> 1)
    y = plsc.bitcast(i, jnp.float32)
    for _ in range(3):
        y = y * (1.5 - (0.5 * x) * y * y)
    return y


def _ln_chunk(buf, obuf, gamma_v, beta_v):
    iota = lax.iota(jnp.int32, 16)
    grp_rows = [iota + 16 * g for g in range(GRPS)]
    zero = jnp.zeros((16,), jnp.float32)

    def pass_a(h, accs):
        col = (h + iota) & (H - 1)
        out = []
        for g in range(GRPS):
            x = plsc.load_gather(buf, [grp_rows[g], col])
            out.append(accs[2 * g] + x)
            out.append(accs[2 * g + 1] + x * x)
        return tuple(out)

    accs = lax.fori_loop(0, H, pass_a, (zero,) * (2 * GRPS))
    means, rstds = [], []
    for g in range(GRPS):
        mean = accs[2 * g] * (1.0 / H)
        var = accs[2 * g + 1] * (1.0 / H) - mean * mean + EPS
        means.append(mean)
        rstds.append(_rsqrt(var))

    def pass_b(h, carry):
        col = (h + iota) & (H - 1)
        gv = plsc.load_gather(gamma_v, [col])
        bv = plsc.load_gather(beta_v, [col])
        for g in range(GRPS):
            x = plsc.load_gather(buf, [grp_rows[g], col])
            y = (x - means[g]) * rstds[g] * gv + bv
            plsc.store_scatter(obuf, [col, grp_rows[g]], y)
        return carry

    lax.fori_loop(0, H, pass_b, 0)


@functools.partial(
    pl.kernel,
    out_type=(
        jax.ShapeDtypeStruct((L, H, B), jnp.float32),
        jax.ShapeDtypeStruct((L, H, B), jnp.float32),
        jax.ShapeDtypeStruct((1, H, B), jnp.float32),
    ),
    mesh=plsc.VectorSubcoreMesh(core_axis_name="c", subcore_axis_name="s"),
    compiler_params=pltpu.CompilerParams(
        use_tc_tiling_on_sc=False, needs_layout_passes=False),
    scratch_types=[
        pltpu.VMEM((2 * B_PW, L), jnp.int32),
        pltpu.VMEM((NCH, B_PW), jnp.int32),
        pltpu.VMEM((B_PW, H), jnp.float32),
        pltpu.VMEM((B_PW, H), jnp.float32),
        pltpu.VMEM((H, B_PW), jnp.float32),
        pltpu.VMEM((H, B_PW), jnp.float32),
        pltpu.VMEM((H,), jnp.float32),
        pltpu.VMEM((H,), jnp.float32),
        pltpu.SemaphoreType.DMA,
        pltpu.SemaphoreType.DMA,
        pltpu.SemaphoreType.DMA,
        pltpu.SemaphoreType.DMA,
    ],
)
def _embed_ln(dx_idx, proc_idx, dx_tab, proc_tab, visit_tab, gamma_b, beta_b,
              dx_out, proc_out, visit_out,
              idx_raw, idx_t, buf0, buf1, obuf0, obuf1, gamma_v, beta_v,
              gsem0, gsem1, fsem0, fsem1):
    wid = lax.axis_index("s") * NC + lax.axis_index("c")
    b_base = wid * B_PW
    pltpu.sync_copy(dx_idx.at[pl.ds(b_base, B_PW)], idx_raw.at[pl.ds(0, B_PW)])
    pltpu.sync_copy(proc_idx.at[pl.ds(b_base, B_PW)],
                    idx_raw.at[pl.ds(B_PW, B_PW)])
    pltpu.sync_copy(gamma_b, gamma_v)
    pltpu.sync_copy(beta_b, beta_v)

    iota = lax.iota(jnp.int32, 16)
    zero_i = jnp.zeros((16,), jnp.int32)

    def build_l(l, carry):
        for t in range(2):
            for g in range(GRPS):
                rows = t * B_PW + 16 * g + iota
                v = plsc.load_gather(idx_raw, [rows, jnp.full((16,), l,
                                                              jnp.int32)])
                idx_t[t * L + l, pl.ds(16 * g, 16)] = v
        return carry

    lax.fori_loop(0, L, build_l, 0)
    for g in range(GRPS):
        idx_t[2 * L, pl.ds(16 * g, 16)] = zero_i

    bufs = (buf0, buf1)
    obufs = (obuf0, obuf1)
    gsems = (gsem0, gsem1)
    fsems = (fsem0, fsem1)

    def gather(j, buf, sem):
        idx = idx_t.at[j]

        @pl.when(j < L)
        def _():
            pltpu.async_copy(dx_tab.at[idx], buf, sem)

        @pl.when(jnp.logical_and(j >= L, j < 2 * L))
        def _():
            pltpu.async_copy(proc_tab.at[idx], buf, sem)

        @pl.when(j >= 2 * L)
        def _():
            pltpu.async_copy(visit_tab.at[idx], buf, sem)

    def gather_drain(buf, sem):
        pltpu.make_async_copy(dx_tab.at[idx_t.at[0]], buf, sem).wait()

    def flush(j, obuf, sem):
        @pl.when(j < L)
        def _():
            pltpu.async_copy(obuf, dx_out.at[j, :, pl.ds(b_base, B_PW)], sem)

        @pl.when(jnp.logical_and(j >= L, j < 2 * L))
        def _():
            pltpu.async_copy(obuf, proc_out.at[j - L, :, pl.ds(b_base, B_PW)],
                             sem)

        @pl.when(j >= 2 * L)
        def _():
            pltpu.async_copy(obuf, visit_out.at[0, :, pl.ds(b_base, B_PW)],
                             sem)

    def flush_drain(obuf, sem):
        pltpu.make_async_copy(obuf, dx_out.at[0, :, pl.ds(b_base, B_PW)],
                              sem).wait()

    gather(0, bufs[0], gsems[0])

    def body(j, carry):
        cur = lax.rem(j, 2)
        for p in range(2):
            @pl.when(cur == p)
            def _():
                gather_drain(bufs[p], gsems[p])

                @pl.when(j + 1 < NCH)
                def _():
                    gather(j + 1, bufs[1 - p], gsems[1 - p])

                @pl.when(j >= 2)
                def _():
                    flush_drain(obufs[p], fsems[p])
                _ln_chunk(bufs[p], obufs[p], gamma_v, beta_v)
                flush(j, obufs[p], fsems[p])
        return carry

    lax.fori_loop(0, NCH, body, 0)
    flush_drain(obufs[0], fsems[0])
    flush_drain(obufs[1], fsems[1])


def kernel(dx_ints1, proc_ints1, number, dx_table, proc_table, visit_table,
           ln_gamma, ln_beta):
    del number
    batch = dx_ints1.shape[0]
    dx_o, proc_o, visit_o = _embed_ln(
        dx_ints1, proc_ints1, dx_table, proc_table, visit_table,
        ln_gamma, ln_beta)
    return (
        jnp.transpose(dx_o, (2, 0, 1)),
        jnp.transpose(proc_o, (2, 0, 1)),
        jnp.transpose(visit_o, (2, 0, 1)),
        jnp.ones((batch, 1), jnp.float32),
    )


# speedup vs baseline: 1.7396x; 1.0048x over previous
import functools

import jax
import jax.numpy as jnp
from jax import lax
from jax.experimental import pallas as pl
from jax.experimental.pallas import tpu as pltpu
from jax.experimental.pallas import tpu_sc as plsc

B, L, H, V = 4096, 50, 64, 1000000
NC, NS = 2, 16
NW = NC * NS
B_PW = B // NW
GRPS = B_PW // 16
NCH = 2 * L + 1
EPS = 1e-5


def _rsqrt(x):
    i = plsc.bitcast(x, jnp.int32)
    i = 0x5F3759DF - (i >> 1)
    y = plsc.bitcast(i, jnp.float32)
    for _ in range(3):
        y = y * (1.5 - (0.5 * x) * y * y)
    return y


def _ln_chunk(buf, obuf, gamma_v, beta_v):
    iota = lax.iota(jnp.int32, 16)
    grp_rows = [iota + 16 * g for g in range(GRPS)]
    zero = jnp.zeros((16,), jnp.float32)

    def pass_a(h, accs):
        col = (h + iota) & (H - 1)
        out = []
        for g in range(GRPS):
            x = plsc.load_gather(buf, [grp_rows[g], col])
            out.append(accs[2 * g] + x)
            out.append(accs[2 * g + 1] + x * x)
        return tuple(out)

    accs = lax.fori_loop(0, H, pass_a, (zero,) * (2 * GRPS))
    means, rstds = [], []
    for g in range(GRPS):
        mean = accs[2 * g] * (1.0 / H)
        var = accs[2 * g + 1] * (1.0 / H) - mean * mean + EPS
        means.append(mean)
        rstds.append(_rsqrt(var))

    def pass_b(h, carry):
        col = (h + iota) & (H - 1)
        gv = plsc.load_gather(gamma_v, [col])
        bv = plsc.load_gather(beta_v, [col])
        for g in range(GRPS):
            x = plsc.load_gather(buf, [grp_rows[g], col])
            y = (x - means[g]) * rstds[g] * gv + bv
            plsc.store_scatter(obuf, [col, grp_rows[g]], y)
        return carry

    lax.fori_loop(0, H, pass_b, 0)


@functools.partial(
    pl.kernel,
    out_type=(
        jax.ShapeDtypeStruct((L, H, B), jnp.float32),
        jax.ShapeDtypeStruct((L, H, B), jnp.float32),
        jax.ShapeDtypeStruct((1, H, B), jnp.float32),
    ),
    mesh=plsc.VectorSubcoreMesh(core_axis_name="c", subcore_axis_name="s"),
    compiler_params=pltpu.CompilerParams(
        use_tc_tiling_on_sc=False, needs_layout_passes=False),
    scratch_types=[
        pltpu.VMEM((2 * B_PW, L), jnp.int32),
        pltpu.VMEM((NCH, B_PW), jnp.int32),
        pltpu.VMEM((B_PW, H), jnp.float32),
        pltpu.VMEM((B_PW, H), jnp.float32),
        pltpu.VMEM((B_PW, H), jnp.float32),
        pltpu.VMEM((B_PW, H), jnp.float32),
        pltpu.VMEM((H, B_PW), jnp.float32),
        pltpu.VMEM((H, B_PW), jnp.float32),
        pltpu.VMEM((H, B_PW), jnp.float32),
        pltpu.VMEM((H, B_PW), jnp.float32),
        pltpu.VMEM((H,), jnp.float32),
        pltpu.VMEM((H,), jnp.float32),
        pltpu.SemaphoreType.DMA,
        pltpu.SemaphoreType.DMA,
        pltpu.SemaphoreType.DMA,
        pltpu.SemaphoreType.DMA,
        pltpu.SemaphoreType.DMA,
        pltpu.SemaphoreType.DMA,
        pltpu.SemaphoreType.DMA,
        pltpu.SemaphoreType.DMA,
    ],
)
def _embed_ln(dx_idx, proc_idx, dx_tab, proc_tab, visit_tab, gamma_b, beta_b,
              dx_out, proc_out, visit_out,
              idx_raw, idx_t, buf0, buf1, buf2, buf3,
              obuf0, obuf1, obuf2, obuf3, gamma_v, beta_v,
              gsem0, gsem1, gsem2, gsem3, fsem0, fsem1, fsem2, fsem3):
    wid = lax.axis_index("s") * NC + lax.axis_index("c")
    b_base = wid * B_PW
    pltpu.sync_copy(dx_idx.at[pl.ds(b_base, B_PW)], idx_raw.at[pl.ds(0, B_PW)])
    pltpu.sync_copy(proc_idx.at[pl.ds(b_base, B_PW)],
                    idx_raw.at[pl.ds(B_PW, B_PW)])
    pltpu.sync_copy(gamma_b, gamma_v)
    pltpu.sync_copy(beta_b, beta_v)

    iota = lax.iota(jnp.int32, 16)
    zero_i = jnp.zeros((16,), jnp.int32)

    def build_l(l, carry):
        for t in range(2):
            for g in range(GRPS):
                rows = t * B_PW + 16 * g + iota
                v = plsc.load_gather(idx_raw, [rows, jnp.full((16,), l,
                                                              jnp.int32)])
                idx_t[t * L + l, pl.ds(16 * g, 16)] = v
        return carry

    lax.fori_loop(0, L, build_l, 0)
    for g in range(GRPS):
        idx_t[2 * L, pl.ds(16 * g, 16)] = zero_i

    bufs = (buf0, buf1, buf2, buf3)
    obufs = (obuf0, obuf1, obuf2, obuf3)
    gsems = (gsem0, gsem1, gsem2, gsem3)
    fsems = (fsem0, fsem1, fsem2, fsem3)

    def gather(j, buf, sem):
        idx = idx_t.at[j]

        @pl.when(j < L)
        def _():
            pltpu.async_copy(dx_tab.at[idx], buf, sem)

        @pl.when(jnp.logical_and(j >= L, j < 2 * L))
        def _():
            pltpu.async_copy(proc_tab.at[idx], buf, sem)

        @pl.when(j >= 2 * L)
        def _():
            pltpu.async_copy(visit_tab.at[idx], buf, sem)

    def gather_drain(buf, sem):
        pltpu.make_async_copy(dx_tab.at[idx_t.at[0]], buf, sem).wait()

    def flush(j, obuf, sem):
        @pl.when(j < L)
        def _():
            pltpu.async_copy(obuf, dx_out.at[j, :, pl.ds(b_base, B_PW)], sem)

        @pl.when(jnp.logical_and(j >= L, j < 2 * L))
        def _():
            pltpu.async_copy(obuf, proc_out.at[j - L, :, pl.ds(b_base, B_PW)],
                             sem)

        @pl.when(j >= 2 * L)
        def _():
            pltpu.async_copy(obuf, visit_out.at[0, :, pl.ds(b_base, B_PW)],
                             sem)

    def flush_drain(obuf, sem):
        pltpu.make_async_copy(obuf, dx_out.at[0, :, pl.ds(b_base, B_PW)],
                              sem).wait()

    for s in range(3):
        gather(s, bufs[s], gsems[s])

    def step(j, s):
        gather_drain(bufs[s], gsems[s])

        @pl.when(j + 3 < NCH)
        def _():
            gather(j + 3, bufs[(s + 3) % 4], gsems[(s + 3) % 4])

        @pl.when(j >= 4)
        def _():
            flush_drain(obufs[s], fsems[s])
        _ln_chunk(bufs[s], obufs[s], gamma_v, beta_v)
        flush(j, obufs[s], fsems[s])

    def body(k, carry):
        for s in range(4):
            step(4 * k + s, s)
        return carry

    lax.fori_loop(0, (NCH - 1) // 4, body, 0)
    step(NCH - 1, 0)
    for s in range(4):
        flush_drain(obufs[s], fsems[s])


def kernel(dx_ints1, proc_ints1, number, dx_table, proc_table, visit_table,
           ln_gamma, ln_beta):
    del number
    batch = dx_ints1.shape[0]
    dx_o, proc_o, visit_o = _embed_ln(
        dx_ints1, proc_ints1, dx_table, proc_table, visit_table,
        ln_gamma, ln_beta)
    return (
        jnp.transpose(dx_o, (2, 0, 1)),
        jnp.transpose(proc_o, (2, 0, 1)),
        jnp.transpose(visit_o, (2, 0, 1)),
        jnp.ones((batch, 1), jnp.float32),
    )


# speedup vs baseline: 1.9554x; 1.1240x over previous
import functools

import jax
import jax.numpy as jnp
from jax import lax
from jax.experimental import pallas as pl
from jax.experimental.pallas import tpu as pltpu
from jax.experimental.pallas import tpu_sc as plsc

B, L, H, V = 4096, 50, 64, 1000000
NC, NS = 2, 16
NW = NC * NS
B_PW = B // NW
GRPS = B_PW // 16
NCH = 2 * L + 1
EPS = 1e-5


def _rsqrt(x):
    i = plsc.bitcast(x, jnp.int32)
    i = 0x5F3759DF - (i >> 1)
    y = plsc.bitcast(i, jnp.float32)
    for _ in range(3):
        y = y * (1.5 - (0.5 * x) * y * y)
    return y


UNROLL = 4


def _ln_chunk(buf, obuf, gsplat, bsplat):
    iota = lax.iota(jnp.int32, 16)
    grp_rows = [iota + 16 * g for g in range(GRPS)]
    zero = jnp.zeros((16,), jnp.float32)

    def pass_a(i, accs):
        out = list(accs)
        for u in range(UNROLL):
            h = UNROLL * i + u
            col = (h + iota) & (H - 1)
            for g in range(GRPS):
                x = plsc.load_gather(buf, [grp_rows[g], col])
                plsc.store_scatter(obuf, [col, grp_rows[g]], x)
                out[2 * g] = out[2 * g] + x
                out[2 * g + 1] = out[2 * g + 1] + x * x
        return tuple(out)

    accs = lax.fori_loop(0, H // UNROLL, pass_a, (zero,) * (2 * GRPS))
    means, rstds = [], []
    for g in range(GRPS):
        mean = accs[2 * g] * (1.0 / H)
        var = accs[2 * g + 1] * (1.0 / H) - mean * mean + EPS
        means.append(mean)
        rstds.append(_rsqrt(var))

    def pass_b(i, carry):
        for u in range(UNROLL):
            h = UNROLL * i + u
            gv = gsplat[h, :]
            bv = bsplat[h, :]
            for g in range(GRPS):
                x = obuf[h, pl.ds(16 * g, 16)]
                y = (x - means[g]) * rstds[g] * gv + bv
                obuf[h, pl.ds(16 * g, 16)] = y
        return carry

    lax.fori_loop(0, H // UNROLL, pass_b, 0)


@functools.partial(
    pl.kernel,
    out_type=(
        jax.ShapeDtypeStruct((L, H, B), jnp.float32),
        jax.ShapeDtypeStruct((L, H, B), jnp.float32),
        jax.ShapeDtypeStruct((1, H, B), jnp.float32),
    ),
    mesh=plsc.VectorSubcoreMesh(core_axis_name="c", subcore_axis_name="s"),
    compiler_params=pltpu.CompilerParams(
        use_tc_tiling_on_sc=False, needs_layout_passes=False),
    scratch_types=[
        pltpu.VMEM((2 * B_PW, L), jnp.int32),
        pltpu.VMEM((NCH, B_PW), jnp.int32),
        pltpu.VMEM((B_PW, H), jnp.float32),
        pltpu.VMEM((B_PW, H), jnp.float32),
        pltpu.VMEM((B_PW, H), jnp.float32),
        pltpu.VMEM((B_PW, H), jnp.float32),
        pltpu.VMEM((H, B_PW), jnp.float32),
        pltpu.VMEM((H, B_PW), jnp.float32),
        pltpu.VMEM((H, B_PW), jnp.float32),
        pltpu.VMEM((H, B_PW), jnp.float32),
        pltpu.VMEM((H,), jnp.float32),
        pltpu.VMEM((H,), jnp.float32),
        pltpu.VMEM((H, 16), jnp.float32),
        pltpu.VMEM((H, 16), jnp.float32),
        pltpu.SemaphoreType.DMA,
        pltpu.SemaphoreType.DMA,
        pltpu.SemaphoreType.DMA,
        pltpu.SemaphoreType.DMA,
        pltpu.SemaphoreType.DMA,
        pltpu.SemaphoreType.DMA,
        pltpu.SemaphoreType.DMA,
        pltpu.SemaphoreType.DMA,
    ],
)
def _embed_ln(dx_idx, proc_idx, dx_tab, proc_tab, visit_tab, gamma_b, beta_b,
              dx_out, proc_out, visit_out,
              idx_raw, idx_t, buf0, buf1, buf2, buf3,
              obuf0, obuf1, obuf2, obuf3, gamma_v, beta_v, gsplat, bsplat,
              gsem0, gsem1, gsem2, gsem3, fsem0, fsem1, fsem2, fsem3):
    wid = lax.axis_index("s") * NC + lax.axis_index("c")
    b_base = wid * B_PW
    pltpu.sync_copy(dx_idx.at[pl.ds(b_base, B_PW)], idx_raw.at[pl.ds(0, B_PW)])
    pltpu.sync_copy(proc_idx.at[pl.ds(b_base, B_PW)],
                    idx_raw.at[pl.ds(B_PW, B_PW)])
    pltpu.sync_copy(gamma_b, gamma_v)
    pltpu.sync_copy(beta_b, beta_v)

    iota = lax.iota(jnp.int32, 16)
    zero_i = jnp.zeros((16,), jnp.int32)

    def build_l(l, carry):
        for t in range(2):
            for g in range(GRPS):
                rows = t * B_PW + 16 * g + iota
                v = plsc.load_gather(idx_raw, [rows, jnp.full((16,), l,
                                                              jnp.int32)])
                idx_t[t * L + l, pl.ds(16 * g, 16)] = v
        return carry

    lax.fori_loop(0, L, build_l, 0)
    for g in range(GRPS):
        idx_t[2 * L, pl.ds(16 * g, 16)] = zero_i

    for q in range(4):
        gq = gamma_v[pl.ds(16 * q, 16)]
        bq = beta_v[pl.ds(16 * q, 16)]
        h_ids = 16 * q + iota
        for c in range(16):
            cols = (c + iota) & 15
            plsc.store_scatter(gsplat, [h_ids, cols], gq)
            plsc.store_scatter(bsplat, [h_ids, cols], bq)

    bufs = (buf0, buf1, buf2, buf3)
    obufs = (obuf0, obuf1, obuf2, obuf3)
    gsems = (gsem0, gsem1, gsem2, gsem3)
    fsems = (fsem0, fsem1, fsem2, fsem3)

    def gather(j, buf, sem):
        idx = idx_t.at[j]

        @pl.when(j < L)
        def _():
            pltpu.async_copy(dx_tab.at[idx], buf, sem)

        @pl.when(jnp.logical_and(j >= L, j < 2 * L))
        def _():
            pltpu.async_copy(proc_tab.at[idx], buf, sem)

        @pl.when(j >= 2 * L)
        def _():
            pltpu.async_copy(visit_tab.at[idx], buf, sem)

    def gather_drain(buf, sem):
        pltpu.make_async_copy(dx_tab.at[idx_t.at[0]], buf, sem).wait()

    def flush(j, obuf, sem):
        @pl.when(j < L)
        def _():
            pltpu.async_copy(obuf, dx_out.at[j, :, pl.ds(b_base, B_PW)], sem)

        @pl.when(jnp.logical_and(j >= L, j < 2 * L))
        def _():
            pltpu.async_copy(obuf, proc_out.at[j - L, :, pl.ds(b_base, B_PW)],
                             sem)

        @pl.when(j >= 2 * L)
        def _():
            pltpu.async_copy(obuf, visit_out.at[0, :, pl.ds(b_base, B_PW)],
                             sem)

    def flush_drain(obuf, sem):
        pltpu.make_async_copy(obuf, dx_out.at[0, :, pl.ds(b_base, B_PW)],
                              sem).wait()

    for s in range(3):
        gather(s, bufs[s], gsems[s])

    def step(j, s):
        gather_drain(bufs[s], gsems[s])

        @pl.when(j + 3 < NCH)
        def _():
            gather(j + 3, bufs[(s + 3) % 4], gsems[(s + 3) % 4])

        @pl.when(j >= 4)
        def _():
            flush_drain(obufs[s], fsems[s])
        _ln_chunk(bufs[s], obufs[s], gsplat, bsplat)
        flush(j, obufs[s], fsems[s])

    def body(k, carry):
        for s in range(4):
            step(4 * k + s, s)
        return carry

    lax.fori_loop(0, (NCH - 1) // 4, body, 0)
    step(NCH - 1, 0)
    for s in range(4):
        flush_drain(obufs[s], fsems[s])


def kernel(dx_ints1, proc_ints1, number, dx_table, proc_table, visit_table,
           ln_gamma, ln_beta):
    del number
    batch = dx_ints1.shape[0]
    dx_o, proc_o, visit_o = _embed_ln(
        dx_ints1, proc_ints1, dx_table, proc_table, visit_table,
        ln_gamma, ln_beta)
    return (
        jnp.transpose(dx_o, (2, 0, 1)),
        jnp.transpose(proc_o, (2, 0, 1)),
        jnp.transpose(visit_o, (2, 0, 1)),
        jnp.ones((batch, 1), jnp.float32),
    )


# speedup vs baseline: 2.2279x; 1.1394x over previous
import functools

import jax
import jax.numpy as jnp
from jax import lax
from jax.experimental import pallas as pl
from jax.experimental.pallas import tpu as pltpu
from jax.experimental.pallas import tpu_sc as plsc

B, L, H, V = 4096, 50, 64, 1000000
NC, NS = 2, 16
NW = NC * NS
B_PW = B // NW
GRPS = B_PW // 16
EPS = 1e-5
UNROLL = 4


def _rsqrt(x):
    i = plsc.bitcast(x, jnp.int32)
    i = 0x5F3759DF - (i >> 1)
    y = plsc.bitcast(i, jnp.float32)
    for _ in range(3):
        y = y * (1.5 - (0.5 * x) * y * y)
    return y


def _ln_chunk(buf, obuf, gsplat, bsplat):
    iota = lax.iota(jnp.int32, 16)
    grp_rows = [iota + 16 * g for g in range(GRPS)]
    zero = jnp.zeros((16,), jnp.float32)

    def pass_a(i, accs):
        out = list(accs)
        for u in range(UNROLL):
            h = UNROLL * i + u
            col = (h + iota) & (H - 1)
            for g in range(GRPS):
                x = plsc.load_gather(buf, [grp_rows[g], col])
                plsc.store_scatter(obuf, [col, grp_rows[g]], x)
                out[2 * g] = out[2 * g] + x
                out[2 * g + 1] = out[2 * g + 1] + x * x
        return tuple(out)

    accs = lax.fori_loop(0, H // UNROLL, pass_a, (zero,) * (2 * GRPS))
    means, rstds = [], []
    for g in range(GRPS):
        mean = accs[2 * g] * (1.0 / H)
        var = accs[2 * g + 1] * (1.0 / H) - mean * mean + EPS
        means.append(mean)
        rstds.append(_rsqrt(var))

    def pass_b(i, carry):
        for u in range(UNROLL):
            h = UNROLL * i + u
            gv = gsplat[h, :]
            bv = bsplat[h, :]
            for g in range(GRPS):
                x = obuf[h, pl.ds(16 * g, 16)]
                y = (x - means[g]) * rstds[g] * gv + bv
                obuf[h, pl.ds(16 * g, 16)] = y
        return carry

    lax.fori_loop(0, H // UNROLL, pass_b, 0)


def _make_embed(with_visit):
    nch = L + 1 if with_visit else L
    out_type = [jax.ShapeDtypeStruct((L, H, B), jnp.float32)]
    if with_visit:
        out_type.append(jax.ShapeDtypeStruct((1, H, B), jnp.float32))

    @functools.partial(
        pl.kernel,
        out_type=tuple(out_type),
        mesh=plsc.VectorSubcoreMesh(core_axis_name="c", subcore_axis_name="s"),
        compiler_params=pltpu.CompilerParams(
            use_tc_tiling_on_sc=False, needs_layout_passes=False),
        scratch_types=[
            pltpu.VMEM((B_PW, L), jnp.int32),
            pltpu.VMEM((nch, B_PW), jnp.int32),
            pltpu.VMEM((B_PW, H), jnp.float32),
            pltpu.VMEM((B_PW, H), jnp.float32),
            pltpu.VMEM((B_PW, H), jnp.float32),
            pltpu.VMEM((B_PW, H), jnp.float32),
            pltpu.VMEM((H, B_PW), jnp.float32),
            pltpu.VMEM((H, B_PW), jnp.float32),
            pltpu.VMEM((H, B_PW), jnp.float32),
            pltpu.VMEM((H, B_PW), jnp.float32),
            pltpu.VMEM((H,), jnp.float32),
            pltpu.VMEM((H,), jnp.float32),
            pltpu.VMEM((H, 16), jnp.float32),
            pltpu.VMEM((H, 16), jnp.float32),
            pltpu.SemaphoreType.DMA,
            pltpu.SemaphoreType.DMA,
            pltpu.SemaphoreType.DMA,
            pltpu.SemaphoreType.DMA,
            pltpu.SemaphoreType.DMA,
            pltpu.SemaphoreType.DMA,
            pltpu.SemaphoreType.DMA,
            pltpu.SemaphoreType.DMA,
        ],
    )
    def _embed(*refs):
        if with_visit:
            (idx_hbm, tab, visit_tab, gamma_b, beta_b, out, visit_out,
             idx_raw, idx_t, buf0, buf1, buf2, buf3,
             obuf0, obuf1, obuf2, obuf3, gamma_v, beta_v, gsplat, bsplat,
             gsem0, gsem1, gsem2, gsem3, fsem0, fsem1, fsem2, fsem3) = refs
        else:
            (idx_hbm, tab, gamma_b, beta_b, out,
             idx_raw, idx_t, buf0, buf1, buf2, buf3,
             obuf0, obuf1, obuf2, obuf3, gamma_v, beta_v, gsplat, bsplat,
             gsem0, gsem1, gsem2, gsem3, fsem0, fsem1, fsem2, fsem3) = refs

        wid = lax.axis_index("s") * NC + lax.axis_index("c")
        b_base = wid * B_PW
        pltpu.sync_copy(idx_hbm.at[pl.ds(b_base, B_PW)], idx_raw)
        pltpu.sync_copy(gamma_b, gamma_v)
        pltpu.sync_copy(beta_b, beta_v)

        iota = lax.iota(jnp.int32, 16)
        zero_i = jnp.zeros((16,), jnp.int32)

        def build_l(l, carry):
            for g in range(GRPS):
                rows = 16 * g + iota
                v = plsc.load_gather(
                    idx_raw, [rows, jnp.full((16,), l, jnp.int32)])
                idx_t[l, pl.ds(16 * g, 16)] = v
            return carry

        lax.fori_loop(0, L, build_l, 0)
        if with_visit:
            for g in range(GRPS):
                idx_t[L, pl.ds(16 * g, 16)] = zero_i

        for q in range(4):
            gq = gamma_v[pl.ds(16 * q, 16)]
            bq = beta_v[pl.ds(16 * q, 16)]
            h_ids = 16 * q + iota
            for c in range(16):
                cols = (c + iota) & 15
                plsc.store_scatter(gsplat, [h_ids, cols], gq)
                plsc.store_scatter(bsplat, [h_ids, cols], bq)

        bufs = (buf0, buf1, buf2, buf3)
        obufs = (obuf0, obuf1, obuf2, obuf3)
        gsems = (gsem0, gsem1, gsem2, gsem3)
        fsems = (fsem0, fsem1, fsem2, fsem3)

        def gather(j, buf, sem):
            idx = idx_t.at[j]
            if with_visit:
                @pl.when(j < L)
                def _():
                    pltpu.async_copy(tab.at[idx], buf, sem)

                @pl.when(j >= L)
                def _():
                    pltpu.async_copy(visit_tab.at[idx], buf, sem)
            else:
                pltpu.async_copy(tab.at[idx], buf, sem)

        def gather_drain(buf, sem):
            pltpu.make_async_copy(tab.at[idx_t.at[0]], buf, sem).wait()

        def flush(j, obuf, sem):
            if with_visit:
                @pl.when(j < L)
                def _():
                    pltpu.async_copy(obuf, out.at[j, :, pl.ds(b_base, B_PW)],
                                     sem)

                @pl.when(j >= L)
                def _():
                    pltpu.async_copy(
                        obuf, visit_out.at[0, :, pl.ds(b_base, B_PW)], sem)
            else:
                pltpu.async_copy(obuf, out.at[j, :, pl.ds(b_base, B_PW)], sem)

        def flush_drain(obuf, sem):
            pltpu.make_async_copy(obuf, out.at[0, :, pl.ds(b_base, B_PW)],
                                  sem).wait()

        for s in range(3):
            gather(s, bufs[s], gsems[s])

        def step(j, s):
            gather_drain(bufs[s], gsems[s])

            @pl.when(j + 3 < nch)
            def _():
                gather(j + 3, bufs[(s + 3) % 4], gsems[(s + 3) % 4])

            @pl.when(j >= 4)
            def _():
                flush_drain(obufs[s], fsems[s])
            _ln_chunk(bufs[s], obufs[s], gsplat, bsplat)
            flush(j, obufs[s], fsems[s])

        def body(k, carry):
            for s in range(4):
                step(4 * k + s, s)
            return carry

        lax.fori_loop(0, nch // 4, body, 0)
        for t in range(nch - nch % 4, nch):
            step(t, t % 4)
        for s in range(4):
            flush_drain(obufs[s], fsems[s])

    return _embed


_embed_dx = _make_embed(with_visit=True)
_embed_proc = _make_embed(with_visit=False)


def kernel(dx_ints1, proc_ints1, number, dx_table, proc_table, visit_table,
           ln_gamma, ln_beta):
    del number
    batch = dx_ints1.shape[0]
    dx_o, visit_o = _embed_dx(dx_ints1, dx_table, visit_table,
                              ln_gamma, ln_beta)
    (proc_o,) = _embed_proc(proc_ints1, proc_table, ln_gamma, ln_beta)
    return (
        jnp.transpose(dx_o, (2, 0, 1)),
        jnp.transpose(proc_o, (2, 0, 1)),
        jnp.transpose(visit_o, (2, 0, 1)),
        jnp.ones((batch, 1), jnp.float32),
    )
